# baseline (speedup 1.0000x reference)
import functools

import jax
import jax.numpy as jnp
from jax import lax
from jax.experimental import pallas as pl
from jax.experimental.pallas import tpu as pltpu
from jax.experimental.pallas import tpu_sc as plsc

_N = 50000
_D = 128
_S = 512
_C = 128
_NFULL = _N // _C
_NCH = _NFULL + 1
_TSTART = _N - _C
_TPAD = _N - _TSTART - (_N - _NFULL * _C)
_NC = 2
_NS = 16
_NW = _NC * _NS
_K = -(-_NCH // _NW)
_NBUF = 6
_PD = 4
_SROWS = _S + 8

_sc_mesh = plsc.VectorSubcoreMesh(core_axis_name="c", subcore_axis_name="s")


@functools.partial(
    pl.kernel,
    out_type=[
        jax.ShapeDtypeStruct((_NC * _S, _D), jnp.float32),
        jax.ShapeDtypeStruct((_NC * _S, _D), jnp.float32),
    ],
    mesh=_sc_mesh,
    scratch_types=[
        pltpu.VMEM((_NBUF, _C, _D), jnp.float32),
        pltpu.VMEM((_NBUF, _C), jnp.int32),
        pltpu.VMEM((_C, _D), jnp.float32),
        pltpu.VMEM((32, _D), jnp.float32),
        pltpu.VMEM_SHARED((_SROWS, _D), jnp.float32),
        pltpu.VMEM_SHARED((_SROWS, _D), jnp.float32),
    ] + [pltpu.SemaphoreType.DMA] * (2 * _NBUF),
)
def _sc_segsum(x_hbm, ids_hbm, sum_out, cnt_out,
               rows_v, idx_v, ones_v, zbuf, acc_s, cnt_s, *sems):
    load_sem = sems[:_NBUF]
    scat_sem = sems[_NBUF:]
    c = lax.axis_index("c")
    s = lax.axis_index("s")
    wid = s * _NC + c

    zeros16 = jnp.zeros((16,), jnp.float32)
    ones16 = jnp.ones((16,), jnp.float32)

    def _zero_row(i, _):
        def _zero_col(j, _):
            zbuf[i, pl.ds(j * 16, 16)] = zeros16
            return 0
        lax.fori_loop(0, _D // 16, _zero_col, 0)
        return 0
    lax.fori_loop(0, 32, _zero_row, 0)

    def _ones_row(i, _):
        def _ones_col(j, _):
            ones_v[i, pl.ds(j * 16, 16)] = ones16
            return 0
        lax.fori_loop(0, _D // 16, _ones_col, 0)
        return 0
    lax.fori_loop(0, _C, _ones_row, 0)

    pltpu.sync_copy(zbuf, acc_s.at[pl.ds(s * 32, 32)])
    pltpu.sync_copy(zbuf, cnt_s.at[pl.ds(s * 32, 32)])
    plsc.subcore_barrier()

    def _starts(k):
        ch = wid + k * _NW
        full = ch < _NFULL
        tail = ch == _NFULL
        xs = jnp.where(full, ch * _C, _TSTART)
        is_ = jnp.where(full, ch * _C, jnp.where(tail, _N, _N + _C))
        return xs, is_

    def _issue_loads(k, b):
        xs, is_ = _starts(k)
        pltpu.async_copy(x_hbm.at[pl.ds(xs, _C)], rows_v.at[b], load_sem[b])
        pltpu.async_copy(ids_hbm.at[pl.ds(is_, _C)], idx_v.at[b], load_sem[b])

    def _wait_loads(k, b):
        xs, is_ = _starts(k)
        pltpu.make_async_copy(x_hbm.at[pl.ds(xs, _C)], rows_v.at[b],
                              load_sem[b]).wait()
        pltpu.make_async_copy(ids_hbm.at[pl.ds(is_, _C)], idx_v.at[b],
                              load_sem[b]).wait()

    def _issue_scatters(b):
        pltpu.async_copy(rows_v.at[b], acc_s.at[idx_v.at[b]], scat_sem[b],
                         add=True)
        pltpu.async_copy(ones_v, cnt_s.at[idx_v.at[b]], scat_sem[b], add=True)

    def _wait_scatters(b):
        pltpu.make_async_copy(rows_v.at[b], acc_s.at[idx_v.at[b]],
                              scat_sem[b]).wait()
        pltpu.make_async_copy(ones_v, cnt_s.at[idx_v.at[b]],
                              scat_sem[b]).wait()

    for j in range(_PD):
        _issue_loads(j, j % _NBUF)
    for k in range(_K):
        b = k % _NBUF
        _wait_loads(k, b)
        _issue_scatters(b)
        j = k + _PD
        if j < _K:
            bj = j % _NBUF
            if j >= _NBUF:
                _wait_scatters(bj)
            _issue_loads(j, bj)
    for j in range(max(0, _K - _NBUF), _K):
        _wait_scatters(j % _NBUF)
    plsc.subcore_barrier()

    base = c * _S + s * 32
    pltpu.sync_copy(acc_s.at[pl.ds(s * 32, 32)], zbuf)
    pltpu.sync_copy(zbuf, sum_out.at[pl.ds(base, 32)])
    pltpu.sync_copy(cnt_s.at[pl.ds(s * 32, 32)], zbuf)
    pltpu.sync_copy(zbuf, cnt_out.at[pl.ds(base, 32)])


def _mlp_body(p_ref, q_ref, w1_ref, b1_ref, g_ref, be_ref, w2_ref, b2_ref, o_ref):
    sums = p_ref[0:_S, :] + p_ref[_S:2 * _S, :]
    counts = q_ref[0:_S, 0:1] + q_ref[_S:2 * _S, 0:1]
    mean = sums / jnp.maximum(counts, 1.0)
    h = lax.dot_general(mean, w1_ref[...], (((1,), (1,)), ((), ())),
                        preferred_element_type=jnp.float32) + b1_ref[...]
    mu = jnp.mean(h, axis=0, keepdims=True)
    var = jnp.mean((h - mu) * (h - mu), axis=0, keepdims=True)
    h = (h - mu) * (g_ref[...] / jnp.sqrt(var + 1e-5)) + be_ref[...]
    h = jnp.maximum(h, 0.0)
    o_ref[...] = lax.dot_general(h, w2_ref[...], (((1,), (1,)), ((), ())),
                                 preferred_element_type=jnp.float32) + b2_ref[...]


_mlp_call = pl.pallas_call(
    _mlp_body,
    out_shape=jax.ShapeDtypeStruct((_S, 6), jnp.float32),
)


def kernel(x, edge_index, edge_attr, u, batch, W1, b1, gamma, beta, W2, b2):
    ids = batch.astype(jnp.int32)
    tail_ids = jnp.concatenate(
        [jnp.full((_TPAD,), _S, jnp.int32), ids[_TSTART + _TPAD:]])
    ids_ext = jnp.concatenate([ids, tail_ids, jnp.full((_C,), _S, jnp.int32)])
    psum, pcnt = _sc_segsum(x, ids_ext)
    return _mlp_call(psum, pcnt, W1, b1.reshape(1, _D), gamma.reshape(1, _D),
                     beta.reshape(1, _D), W2, b2.reshape(1, 6))


# speedup vs baseline: 1.1313x; 1.1313x over previous
import functools

import jax
import jax.numpy as jnp
from jax import lax
from jax.experimental import pallas as pl
from jax.experimental.pallas import tpu as pltpu
from jax.experimental.pallas import tpu_sc as plsc

_N = 50000
_D = 128
_S = 512
_C = 80
_NCH = _N // _C
_NC = 2
_NS = 16
_NW = _NC * _NS
_K = -(-_NCH // _NW)
_NBUF = 8
_PD = 4
_SROWS = _S + 8

_sc_mesh = plsc.VectorSubcoreMesh(core_axis_name="c", subcore_axis_name="s")


@functools.partial(
    pl.kernel,
    out_type=[
        jax.ShapeDtypeStruct((_NC * _S, _D), jnp.float32),
        jax.ShapeDtypeStruct((_NC * _S, _D), jnp.float32),
    ],
    mesh=_sc_mesh,
    scratch_types=[
        pltpu.VMEM((_NBUF, _C, _D), jnp.float32),
        pltpu.VMEM((_NBUF, _C), jnp.int32),
        pltpu.VMEM((_C, _D), jnp.float32),
        pltpu.VMEM((32, _D), jnp.float32),
        pltpu.VMEM_SHARED((_SROWS, _D), jnp.float32),
        pltpu.VMEM_SHARED((_SROWS, _D), jnp.float32),
    ] + [pltpu.SemaphoreType.DMA] * (2 * _NBUF),
)
def _sc_segsum(x_hbm, ids_hbm, sum_out, cnt_out,
               rows_v, idx_v, ones_v, zbuf, acc_s, cnt_s, *sems):
    load_sem = sems[:_NBUF]
    scat_sem = sems[_NBUF:]
    c = lax.axis_index("c")
    s = lax.axis_index("s")
    wid = s * _NC + c

    zeros16 = jnp.zeros((16,), jnp.float32)
    ones16 = jnp.ones((16,), jnp.float32)

    def _zero_row(i, _):
        def _zero_col(j, _):
            zbuf[i, pl.ds(j * 16, 16)] = zeros16
            return 0
        lax.fori_loop(0, _D // 16, _zero_col, 0)
        return 0
    lax.fori_loop(0, 32, _zero_row, 0)

    def _ones_row(i, _):
        def _ones_col(j, _):
            ones_v[i, pl.ds(j * 16, 16)] = ones16
            return 0
        lax.fori_loop(0, _D // 16, _ones_col, 0)
        return 0
    lax.fori_loop(0, _C, _ones_row, 0)

    pltpu.sync_copy(zbuf, acc_s.at[pl.ds(s * 32, 32)])
    pltpu.sync_copy(zbuf, cnt_s.at[pl.ds(s * 32, 32)])
    plsc.subcore_barrier()

    def _starts(k):
        ch = wid + k * _NW
        valid = ch < _NCH
        xs = jnp.where(valid, ch, 0) * _C
        is_ = jnp.where(valid, ch * _C, _N)
        return xs, is_

    def _issue_loads(k, b):
        xs, is_ = _starts(k)
        pltpu.async_copy(x_hbm.at[pl.ds(xs, _C)], rows_v.at[b], load_sem[b])
        pltpu.async_copy(ids_hbm.at[pl.ds(is_, _C)], idx_v.at[b], load_sem[b])

    def _wait_loads(k, b):
        xs, is_ = _starts(k)
        pltpu.make_async_copy(x_hbm.at[pl.ds(xs, _C)], rows_v.at[b],
                              load_sem[b]).wait()
        pltpu.make_async_copy(ids_hbm.at[pl.ds(is_, _C)], idx_v.at[b],
                              load_sem[b]).wait()

    def _issue_scatters(b):
        pltpu.async_copy(rows_v.at[b], acc_s.at[idx_v.at[b]], scat_sem[b],
                         add=True)
        pltpu.async_copy(ones_v, cnt_s.at[idx_v.at[b]], scat_sem[b], add=True)

    def _wait_scatters(b):
        pltpu.make_async_copy(rows_v.at[b], acc_s.at[idx_v.at[b]],
                              scat_sem[b]).wait()
        pltpu.make_async_copy(ones_v, cnt_s.at[idx_v.at[b]],
                              scat_sem[b]).wait()

    for j in range(_PD):
        _issue_loads(j, j % _NBUF)
    for k in range(_K):
        b = k % _NBUF
        _wait_loads(k, b)
        _issue_scatters(b)
        j = k + _PD
        if j < _K:
            bj = j % _NBUF
            if j >= _NBUF:
                _wait_scatters(bj)
            _issue_loads(j, bj)
    for j in range(max(0, _K - _NBUF), _K):
        _wait_scatters(j % _NBUF)
    plsc.subcore_barrier()

    base = c * _S + s * 32
    pltpu.sync_copy(acc_s.at[pl.ds(s * 32, 32)], zbuf)
    pltpu.sync_copy(zbuf, sum_out.at[pl.ds(base, 32)])
    pltpu.sync_copy(cnt_s.at[pl.ds(s * 32, 32)], zbuf)
    pltpu.sync_copy(zbuf, cnt_out.at[pl.ds(base, 32)])


def _mlp_body(p_ref, q_ref, w1_ref, b1_ref, g_ref, be_ref, w2_ref, b2_ref, o_ref):
    sums = p_ref[0:_S, :] + p_ref[_S:2 * _S, :]
    counts = q_ref[0:_S, 0:1] + q_ref[_S:2 * _S, 0:1]
    mean = sums / jnp.maximum(counts, 1.0)
    h = lax.dot_general(mean, w1_ref[...], (((1,), (1,)), ((), ())),
                        preferred_element_type=jnp.float32) + b1_ref[...]
    mu = jnp.mean(h, axis=0, keepdims=True)
    var = jnp.mean((h - mu) * (h - mu), axis=0, keepdims=True)
    h = (h - mu) * (g_ref[...] / jnp.sqrt(var + 1e-5)) + be_ref[...]
    h = jnp.maximum(h, 0.0)
    o_ref[...] = lax.dot_general(h, w2_ref[...], (((1,), (1,)), ((), ())),
                                 preferred_element_type=jnp.float32) + b2_ref[...]


_mlp_call = pl.pallas_call(
    _mlp_body,
    out_shape=jax.ShapeDtypeStruct((_S, 6), jnp.float32),
)


def kernel(x, edge_index, edge_attr, u, batch, W1, b1, gamma, beta, W2, b2):
    ids = batch.astype(jnp.int32)
    ids_ext = jnp.concatenate([ids, jnp.full((_C,), _S, jnp.int32)])
    psum, pcnt = _sc_segsum(x, ids_ext)
    return _mlp_call(psum, pcnt, W1, b1.reshape(1, _D), gamma.reshape(1, _D),
                     beta.reshape(1, _D), W2, b2.reshape(1, 6))
